# Initial kernel scaffold; baseline (speedup 1.0000x reference)
import jax
import jax.numpy as jnp
from jax import lax
from jax.experimental import pallas as pl
from jax.experimental.pallas import tpu as pltpu
from jax.experimental.pallas import tpu_sc as plsc

N = 10000
E = 320000
D = 128
D_OUT = 64
G = 64
EPS = 1e-5

NC = 2
NS = 16
NW = NC * NS
EPW = E // NW
CH = 128
NCH = 80
NCHT = NCH + 2
ACC_N = 10240
RPT = ACC_N // NS
ZR = 64
BLK = 1000
HIGH = lax.Precision.HIGHEST


def _sc_degree_body(dst_hbm, out_hbm, dstb, hist, sem):
    c = lax.axis_index("c")
    s = lax.axis_index("s")
    wid = c * NS + s
    pltpu.async_copy(dst_hbm.at[wid], dstb, sem).wait()

    zeros16 = jnp.zeros((16,), jnp.float32)

    def zero_body(i, carry):
        hist[pl.ds(i * 16, 16)] = zeros16
        return carry

    lax.fori_loop(0, ACC_N // 16, zero_body, 0)

    ones16 = jnp.ones((16,), jnp.float32)

    def hist_body(j, carry):
        for i in range(CH // 16):
            idx = dstb[j, pl.ds(i * 16, 16)]
            plsc.addupdate_scatter(hist, [idx], ones16)
        return carry

    lax.fori_loop(0, NCH, hist_body, 0)
    pltpu.sync_copy(hist, out_hbm.at[wid])


def _sc_degree(dst_w):
    return pl.kernel(
        _sc_degree_body,
        out_type=jax.ShapeDtypeStruct((NW, ACC_N), jnp.float32),
        mesh=plsc.VectorSubcoreMesh(core_axis_name="c", subcore_axis_name="s"),
        scratch_types=[
            pltpu.VMEM((NCHT, CH), jnp.int32),
            pltpu.VMEM((ACC_N,), jnp.float32),
            pltpu.SemaphoreType.DMA,
        ],
    )(dst_w)


def _sc_scatter_body(hd_hbm, src_hbm, dst_hbm, out_hbm,
                     srcb, dstb, rows0, rows1, zbuf, acc,
                     semi, sem0, sem1):
    c = lax.axis_index("c")
    s = lax.axis_index("s")
    wid = c * NS + s

    cp_s = pltpu.async_copy(src_hbm.at[wid], srcb, semi)
    cp_d = pltpu.async_copy(dst_hbm.at[wid], dstb, semi)

    zeros16 = jnp.zeros((16,), jnp.float32)

    def zb_body(i, carry):
        for k in range(D // 16):
            zbuf[i, pl.ds(k * 16, 16)] = zeros16
        return carry

    lax.fori_loop(0, ZR, zb_body, 0)

    def zacc_body(t, carry):
        pltpu.sync_copy(zbuf, acc.at[pl.ds(s * RPT + t * ZR, ZR)])
        return carry

    lax.fori_loop(0, RPT // ZR, zacc_body, 0)
    cp_s.wait()
    cp_d.wait()
    plsc.subcore_barrier()

    def gdesc(j, rows, sem):
        return pltpu.make_async_copy(hd_hbm.at[srcb.at[j]], rows, sem)

    gdesc(0, rows0, sem0).start()
    gdesc(1, rows1, sem1).start()

    def loop_body(jj, carry):
        j0 = 2 * jj
        gdesc(j0, rows0, sem0).wait()
        pltpu.sync_copy(rows0, acc.at[dstb.at[j0]], add=True)
        gdesc(j0 + 2, rows0, sem0).start()
        j1 = j0 + 1
        gdesc(j1, rows1, sem1).wait()
        pltpu.sync_copy(rows1, acc.at[dstb.at[j1]], add=True)
        gdesc(j1 + 2, rows1, sem1).start()
        return carry

    lax.fori_loop(0, NCH // 2, loop_body, 0)
    gdesc(NCH, rows0, sem0).wait()
    gdesc(NCH + 1, rows1, sem1).wait()

    plsc.subcore_barrier()
    pltpu.sync_copy(acc.at[pl.ds(s * RPT, RPT)],
                    out_hbm.at[c, pl.ds(s * RPT, RPT)])


def _sc_scatter(hd, src_w, dst_w):
    return pl.kernel(
        _sc_scatter_body,
        out_type=jax.ShapeDtypeStruct((NC, ACC_N, D), jnp.float32),
        mesh=plsc.VectorSubcoreMesh(core_axis_name="c", subcore_axis_name="s"),
        scratch_types=[
            pltpu.VMEM((NCHT, CH), jnp.int32),
            pltpu.VMEM((NCHT, CH), jnp.int32),
            pltpu.VMEM((CH, D), jnp.float32),
            pltpu.VMEM((CH, D), jnp.float32),
            pltpu.VMEM((ZR, D), jnp.float32),
            pltpu.VMEM_SHARED((ACC_N, D), jnp.float32),
            pltpu.SemaphoreType.DMA,
            pltpu.SemaphoreType.DMA,
            pltpu.SemaphoreType.DMA,
        ],
    )(hd, src_w, dst_w)


def _dis_from_hists(hists_blk):
    deg = jnp.sum(hists_blk, axis=0) + 1.0
    return lax.rsqrt(deg)


def _tc_hd1_body(x_ref, w1_ref, hists_ref, hd1_ref):
    dis = _dis_from_hists(hists_ref[...])
    h = jnp.dot(x_ref[...], w1_ref[...], preferred_element_type=jnp.float32,
                precision=HIGH)
    hd1_ref[...] = h * dis[:, None]


def _tc_hd1(x, w1, hists):
    return pl.pallas_call(
        _tc_hd1_body,
        grid=(N // BLK,),
        in_specs=[
            pl.BlockSpec((BLK, D), lambda i: (i, 0)),
            pl.BlockSpec((D, D), lambda i: (0, 0)),
            pl.BlockSpec((NW, BLK), lambda i: (0, i)),
        ],
        out_specs=pl.BlockSpec((BLK, D), lambda i: (i, 0)),
        out_shape=jax.ShapeDtypeStruct((N, D), jnp.float32),
    )(x, w1, hists)


def _tc_layer_body(acc_ref, hd_ref, hists_ref, b_ref, g_ref, be_ref, w_ref,
                   h_ref, hdn_ref):
    dis = _dis_from_hists(hists_ref[...])
    agg = acc_ref[0] + acc_ref[1] + hd_ref[...]
    out = dis[:, None] * agg + b_ref[0, :]
    bnscale = g_ref[0, :] / jnp.sqrt(1.0 + EPS)
    h = jnp.maximum(out * bnscale + be_ref[0, :], 0.0)
    h_ref[...] = h
    hdn = jnp.dot(h, w_ref[...], preferred_element_type=jnp.float32,
                  precision=HIGH)
    hdn_ref[...] = hdn * dis[:, None]


def _tc_layer(acc, hd, hists, b, g, be, w):
    return pl.pallas_call(
        _tc_layer_body,
        grid=(N // BLK,),
        in_specs=[
            pl.BlockSpec((NC, BLK, D), lambda i: (0, i, 0)),
            pl.BlockSpec((BLK, D), lambda i: (i, 0)),
            pl.BlockSpec((NW, BLK), lambda i: (0, i)),
            pl.BlockSpec((1, D), lambda i: (0, 0)),
            pl.BlockSpec((1, D), lambda i: (0, 0)),
            pl.BlockSpec((1, D), lambda i: (0, 0)),
            pl.BlockSpec((D, D), lambda i: (0, 0)),
        ],
        out_specs=[
            pl.BlockSpec((BLK, D), lambda i: (i, 0)),
            pl.BlockSpec((BLK, D), lambda i: (i, 0)),
        ],
        out_shape=[
            jax.ShapeDtypeStruct((N, D), jnp.float32),
            jax.ShapeDtypeStruct((N, D), jnp.float32),
        ],
    )(acc, hd, hists, b, g, be, w)


def _tc_final_body(acc_ref, hd_ref, hists_ref, b_ref, g_ref, be_ref,
                   x_ref, h1_ref, batch_ref,
                   p0_ref, p1_ref, p2_ref, pb0_ref, pb1_ref, pb2_ref,
                   score_ref):
    dis = _dis_from_hists(hists_ref[...])
    agg = acc_ref[0] + acc_ref[1] + hd_ref[...]
    out = dis[:, None] * agg + b_ref[0, :]
    bnscale = g_ref[0, :] / jnp.sqrt(1.0 + EPS)
    h2 = jnp.maximum(out * bnscale + be_ref[0, :], 0.0)

    t = jnp.dot(x_ref[...], p0_ref[...], preferred_element_type=jnp.float32,
                precision=HIGH)
    t += jnp.dot(h1_ref[...], p1_ref[...], preferred_element_type=jnp.float32,
                 precision=HIGH)
    t += jnp.dot(h2, p2_ref[...], preferred_element_type=jnp.float32,
                 precision=HIGH)

    b = batch_ref[0, 0, :]
    gio = lax.broadcasted_iota(jnp.int32, (BLK, G), 1)
    onehot = (b[:, None] == gio).astype(jnp.float32)
    contrib = lax.dot_general(onehot, t, (((0,), (0,)), ((), ())),
                              preferred_element_type=jnp.float32,
                              precision=HIGH)

    @pl.when(pl.program_id(0) == 0)
    def _():
        pbs = pb0_ref[0, :] + pb1_ref[0, :] + pb2_ref[0, :]
        score_ref[...] = jnp.broadcast_to(pbs[None, :], (G, D_OUT))

    score_ref[...] += contrib


def _tc_final(acc, hd, hists, b, g, be, x, h1, batch_r, p0, p1, p2,
              pb0, pb1, pb2):
    return pl.pallas_call(
        _tc_final_body,
        grid=(N // BLK,),
        in_specs=[
            pl.BlockSpec((NC, BLK, D), lambda i: (0, i, 0)),
            pl.BlockSpec((BLK, D), lambda i: (i, 0)),
            pl.BlockSpec((NW, BLK), lambda i: (0, i)),
            pl.BlockSpec((1, D), lambda i: (0, 0)),
            pl.BlockSpec((1, D), lambda i: (0, 0)),
            pl.BlockSpec((1, D), lambda i: (0, 0)),
            pl.BlockSpec((BLK, D), lambda i: (i, 0)),
            pl.BlockSpec((BLK, D), lambda i: (i, 0)),
            pl.BlockSpec((1, 1, BLK), lambda i: (i, 0, 0)),
            pl.BlockSpec((D, D_OUT), lambda i: (0, 0)),
            pl.BlockSpec((D, D_OUT), lambda i: (0, 0)),
            pl.BlockSpec((D, D_OUT), lambda i: (0, 0)),
            pl.BlockSpec((1, D_OUT), lambda i: (0, 0)),
            pl.BlockSpec((1, D_OUT), lambda i: (0, 0)),
            pl.BlockSpec((1, D_OUT), lambda i: (0, 0)),
        ],
        out_specs=pl.BlockSpec((G, D_OUT), lambda i: (0, 0)),
        out_shape=jax.ShapeDtypeStruct((G, D_OUT), jnp.float32),
    )(acc, hd, hists, b, g, be, x, h1, batch_r, p0, p1, p2, pb0, pb1, pb2)


def kernel(x, edge_index, batch, W1, b1, g1, be1, W2, b2, g2, be2,
           P0, pb0, P1, pb1, P2, pb2):
    src, dst = edge_index[0], edge_index[1]
    pad = NCHT * CH - EPW
    src_w = jnp.pad(src.reshape(NW, EPW), ((0, 0), (0, pad))).reshape(NW, NCHT, CH)
    dst_w = jnp.pad(dst.reshape(NW, EPW), ((0, 0), (0, pad)),
                    constant_values=N).reshape(NW, NCHT, CH)

    hists = _sc_degree(dst_w)

    hd1 = _tc_hd1(x, W1, hists)
    acc1 = _sc_scatter(hd1, src_w, dst_w)
    h1, hd2 = _tc_layer(acc1, hd1, hists,
                        b1.reshape(1, D), g1.reshape(1, D), be1.reshape(1, D),
                        W2)
    acc2 = _sc_scatter(hd2, src_w, dst_w)

    batch_r = batch.reshape(N // BLK, 1, BLK)
    score = _tc_final(acc2, hd2, hists,
                      b2.reshape(1, D), g2.reshape(1, D), be2.reshape(1, D),
                      x, h1, batch_r,
                      P0, P1, P2,
                      pb0.reshape(1, D_OUT), pb1.reshape(1, D_OUT),
                      pb2.reshape(1, D_OUT))
    return score


# speedup vs baseline: 4.3694x; 4.3694x over previous
import jax
import jax.numpy as jnp
from jax import lax
from jax.experimental import pallas as pl
from jax.experimental.pallas import tpu as pltpu
from jax.experimental.pallas import tpu_sc as plsc

N = 10000
E = 320000
D = 128
D_OUT = 64
G = 64
EPS = 1e-5

NC = 2
NS = 16
NW = NC * NS
EPT = E // NS
CH = 32
GS = 16
NG = 40
NGT = NG + 2
NCHT = NGT * GS
HRNG = N // NC
ACC_H = 5120
DUMMY = 5100
RPT = ACC_H // NS
ZR = CH
DGH = NCHT // 2
DGG = 4
BLK = 1000
HIGH = lax.Precision.HIGHEST


N_HIST = 10048


def _sc_degree_body(dst_hbm, out_hbm, dstc, hist, sem):
    c = lax.axis_index("c")
    s = lax.axis_index("s")
    wid = c * NS + s
    row = wid // 2
    half = wid % 2

    zeros16 = jnp.zeros((16,), jnp.float32)

    def zero_body(i, carry):
        hist[pl.ds(i * 16, 16)] = zeros16
        return carry

    lax.fori_loop(0, N_HIST // 16, zero_body, 0)

    ones16 = jnp.ones((16,), jnp.float32)

    def hist_body(g, carry):
        pltpu.async_copy(
            dst_hbm.at[row, pl.ds(half * DGH + g * DGG, DGG)], dstc, sem
        ).wait()
        for r in range(DGG):
            for i in range(CH // 16):
                idx = dstc[r, pl.ds(i * 16, 16)]
                plsc.addupdate_scatter(hist, [idx], ones16)
        return carry

    lax.fori_loop(0, DGH // DGG, hist_body, 0)
    pltpu.sync_copy(hist, out_hbm.at[wid])


def _sc_degree(dst_w):
    return pl.kernel(
        _sc_degree_body,
        out_type=jax.ShapeDtypeStruct((NW, N_HIST), jnp.float32),
        mesh=plsc.VectorSubcoreMesh(core_axis_name="c", subcore_axis_name="s"),
        scratch_types=[
            pltpu.VMEM((DGG, CH), jnp.int32),
            pltpu.VMEM((N_HIST,), jnp.float32),
            pltpu.SemaphoreType.DMA,
        ],
        compiler_params=pltpu.CompilerParams(needs_layout_passes=False),
    )(dst_w)


def _sc_scatter_body(hd_hbm, src_hbm, dst_hbm, out_hbm,
                     srcg, dstg, rows0, rows1, acc,
                     semg0, semg1, sem0, sem1):
    c = lax.axis_index("c")
    s = lax.axis_index("s")
    base = c * HRNG

    zeros16 = jnp.zeros((16,), jnp.float32)

    def zb_body(i, carry):
        for k in range(D // 16):
            rows0[i, pl.ds(k * 16, 16)] = zeros16
        return carry

    lax.fori_loop(0, ZR, zb_body, 0)

    def zacc_body(t, carry):
        pltpu.sync_copy(rows0, acc.at[pl.ds(s * RPT + t * ZR, ZR)])
        return carry

    lax.fori_loop(0, RPT // ZR, zacc_body, 0)
    plsc.subcore_barrier()

    def idesc(g, slot, sem):
        return (
            pltpu.make_async_copy(src_hbm.at[s, pl.ds(g * GS, GS)],
                                  srcg.at[slot], sem),
            pltpu.make_async_copy(dst_hbm.at[s, pl.ds(g * GS, GS)],
                                  dstg.at[slot], sem),
        )

    def istart(g, slot, sem):
        a, b = idesc(g, slot, sem)
        a.start()
        b.start()

    def iwait(g, slot, sem):
        a, b = idesc(g, slot, sem)
        a.wait()
        b.wait()

    def remap(slot):
        def rm_body(r, carry):
            for i in range(CH // 16):
                v = dstg[slot, r, pl.ds(i * 16, 16)]
                t = v - base
                ok = (t >= 0) & (t < HRNG)
                dstg[slot, r, pl.ds(i * 16, 16)] = jnp.where(ok, t, DUMMY)
            return carry

        lax.fori_loop(0, GS, rm_body, 0)

    def gdesc(j, rows, sem):
        slot = (j // GS) % 2
        return pltpu.make_async_copy(hd_hbm.at[srcg.at[slot, j % GS]],
                                     rows, sem)

    istart(0, 0, semg0)
    iwait(0, 0, semg0)
    remap(0)
    istart(1, 1, semg1)
    gdesc(0, rows0, sem0).start()
    gdesc(1, rows1, sem1).start()

    def outer2(gg, carry):
        g0 = 2 * gg
        a, b = idesc(g0 + 1, 1, semg1)
        a.wait()
        b.wait()
        remap(1)

        def inner(jj, carry2):
            j = g0 * GS + 2 * jj
            gdesc(j, rows0, sem0).wait()
            pltpu.sync_copy(rows0, acc.at[dstg.at[(j // GS) % 2, j % GS]],
                            add=True)
            gdesc(j + 2, rows0, sem0).start()
            j1 = j + 1
            gdesc(j1, rows1, sem1).wait()
            pltpu.sync_copy(rows1, acc.at[dstg.at[(j1 // GS) % 2, j1 % GS]],
                            add=True)
            gdesc(j1 + 2, rows1, sem1).start()
            return carry2

        lax.fori_loop(0, GS // 2, inner, 0)
        istart(g0 + 2, 0, semg0)

        a, b = idesc(g0 + 2, 0, semg0)
        a.wait()
        b.wait()
        remap(0)

        def inner2(jj, carry2):
            j = (g0 + 1) * GS + 2 * jj
            gdesc(j, rows0, sem0).wait()
            pltpu.sync_copy(rows0, acc.at[dstg.at[(j // GS) % 2, j % GS]],
                            add=True)
            gdesc(j + 2, rows0, sem0).start()
            j1 = j + 1
            gdesc(j1, rows1, sem1).wait()
            pltpu.sync_copy(rows1, acc.at[dstg.at[(j1 // GS) % 2, j1 % GS]],
                            add=True)
            gdesc(j1 + 2, rows1, sem1).start()
            return carry2

        lax.fori_loop(0, GS // 2, inner2, 0)
        istart(g0 + 3, 1, semg1)
        return carry

    lax.fori_loop(0, NG // 2, outer2, 0)

    a, b = idesc(NG + 1, 1, semg1)
    a.wait()
    b.wait()
    gdesc(NG * GS, rows0, sem0).wait()
    gdesc(NG * GS + 1, rows1, sem1).wait()

    plsc.subcore_barrier()
    pltpu.sync_copy(acc.at[pl.ds(s * RPT, RPT)],
                    out_hbm.at[c, pl.ds(s * RPT, RPT)])


def _sc_scatter(hd, src_w, dst_w):
    return pl.kernel(
        _sc_scatter_body,
        out_type=jax.ShapeDtypeStruct((NC, ACC_H, D), jnp.float32),
        mesh=plsc.VectorSubcoreMesh(core_axis_name="c", subcore_axis_name="s"),
        scratch_types=[
            pltpu.VMEM((2, GS, CH), jnp.int32),
            pltpu.VMEM((2, GS, CH), jnp.int32),
            pltpu.VMEM((CH, D), jnp.float32),
            pltpu.VMEM((CH, D), jnp.float32),
            pltpu.VMEM_SHARED((ACC_H, D), jnp.float32),
            pltpu.SemaphoreType.DMA,
            pltpu.SemaphoreType.DMA,
            pltpu.SemaphoreType.DMA,
            pltpu.SemaphoreType.DMA,
        ],
    )(hd, src_w, dst_w)


def _tc_dis_body(hists_ref, dis_ref):
    deg = jnp.sum(hists_ref[...], axis=0) + 1.0
    dis_ref[...] = lax.rsqrt(deg)[:, None]


def _tc_dis(hists):
    return pl.pallas_call(
        _tc_dis_body,
        out_shape=jax.ShapeDtypeStruct((N_HIST, 1), jnp.float32),
    )(hists)


def _tc_hd1_body(x_ref, w1_ref, dis_ref, hd1_ref):
    h = jnp.dot(x_ref[...], w1_ref[...], preferred_element_type=jnp.float32,
                precision=HIGH)
    hd1_ref[...] = h * dis_ref[...]


def _tc_hd1(x, w1, dis):
    return pl.pallas_call(
        _tc_hd1_body,
        grid=(N // BLK,),
        in_specs=[
            pl.BlockSpec((BLK, D), lambda i: (i, 0)),
            pl.BlockSpec((D, D), lambda i: (0, 0)),
            pl.BlockSpec((BLK, 1), lambda i: (i, 0)),
        ],
        out_specs=pl.BlockSpec((BLK, D), lambda i: (i, 0)),
        out_shape=jax.ShapeDtypeStruct((N, D), jnp.float32),
    )(x, w1, dis)


def _acc_spec():
    return pl.BlockSpec((1, BLK, D), lambda i: (i // 5, i % 5, 0))


def _tc_layer_body(acc_ref, hd_ref, dis_ref, b_ref, g_ref, be_ref, w_ref,
                   h_ref, hdn_ref):
    dis = dis_ref[...]
    agg = acc_ref[0] + hd_ref[...]
    out = dis * agg + b_ref[0, :]
    bnscale = g_ref[0, :] / jnp.sqrt(1.0 + EPS)
    h = jnp.maximum(out * bnscale + be_ref[0, :], 0.0)
    h_ref[...] = h
    hdn = jnp.dot(h, w_ref[...], preferred_element_type=jnp.float32,
                  precision=HIGH)
    hdn_ref[...] = hdn * dis


def _tc_layer(acc, hd, dis, b, g, be, w):
    return pl.pallas_call(
        _tc_layer_body,
        grid=(N // BLK,),
        in_specs=[
            _acc_spec(),
            pl.BlockSpec((BLK, D), lambda i: (i, 0)),
            pl.BlockSpec((BLK, 1), lambda i: (i, 0)),
            pl.BlockSpec((1, D), lambda i: (0, 0)),
            pl.BlockSpec((1, D), lambda i: (0, 0)),
            pl.BlockSpec((1, D), lambda i: (0, 0)),
            pl.BlockSpec((D, D), lambda i: (0, 0)),
        ],
        out_specs=[
            pl.BlockSpec((BLK, D), lambda i: (i, 0)),
            pl.BlockSpec((BLK, D), lambda i: (i, 0)),
        ],
        out_shape=[
            jax.ShapeDtypeStruct((N, D), jnp.float32),
            jax.ShapeDtypeStruct((N, D), jnp.float32),
        ],
    )(acc, hd, dis, b, g, be, w)


def _tc_final_body(acc_ref, hd_ref, dis_ref, b_ref, g_ref, be_ref,
                   x_ref, h1_ref, batch_ref,
                   p0_ref, p1_ref, p2_ref, pb0_ref, pb1_ref, pb2_ref,
                   score_ref):
    agg = acc_ref[0] + hd_ref[...]
    out = dis_ref[...] * agg + b_ref[0, :]
    bnscale = g_ref[0, :] / jnp.sqrt(1.0 + EPS)
    h2 = jnp.maximum(out * bnscale + be_ref[0, :], 0.0)

    t = jnp.dot(x_ref[...], p0_ref[...], preferred_element_type=jnp.float32,
                precision=HIGH)
    t += jnp.dot(h1_ref[...], p1_ref[...], preferred_element_type=jnp.float32,
                 precision=HIGH)
    t += jnp.dot(h2, p2_ref[...], preferred_element_type=jnp.float32,
                 precision=HIGH)

    b = batch_ref[0, 0, :]
    gio = lax.broadcasted_iota(jnp.int32, (BLK, G), 1)
    onehot = (b[:, None] == gio).astype(jnp.float32)
    contrib = lax.dot_general(onehot, t, (((0,), (0,)), ((), ())),
                              preferred_element_type=jnp.float32,
                              precision=HIGH)

    @pl.when(pl.program_id(0) == 0)
    def _():
        pbs = pb0_ref[0, :] + pb1_ref[0, :] + pb2_ref[0, :]
        score_ref[...] = jnp.broadcast_to(pbs[None, :], (G, D_OUT))

    score_ref[...] += contrib


def _tc_final(acc, hd, dis, b, g, be, x, h1, batch_r, p0, p1, p2,
              pb0, pb1, pb2):
    return pl.pallas_call(
        _tc_final_body,
        grid=(N // BLK,),
        in_specs=[
            _acc_spec(),
            pl.BlockSpec((BLK, D), lambda i: (i, 0)),
            pl.BlockSpec((BLK, 1), lambda i: (i, 0)),
            pl.BlockSpec((1, D), lambda i: (0, 0)),
            pl.BlockSpec((1, D), lambda i: (0, 0)),
            pl.BlockSpec((1, D), lambda i: (0, 0)),
            pl.BlockSpec((BLK, D), lambda i: (i, 0)),
            pl.BlockSpec((BLK, D), lambda i: (i, 0)),
            pl.BlockSpec((1, 1, BLK), lambda i: (i, 0, 0)),
            pl.BlockSpec((D, D_OUT), lambda i: (0, 0)),
            pl.BlockSpec((D, D_OUT), lambda i: (0, 0)),
            pl.BlockSpec((D, D_OUT), lambda i: (0, 0)),
            pl.BlockSpec((1, D_OUT), lambda i: (0, 0)),
            pl.BlockSpec((1, D_OUT), lambda i: (0, 0)),
            pl.BlockSpec((1, D_OUT), lambda i: (0, 0)),
        ],
        out_specs=pl.BlockSpec((G, D_OUT), lambda i: (0, 0)),
        out_shape=jax.ShapeDtypeStruct((G, D_OUT), jnp.float32),
    )(acc, hd, dis, b, g, be, x, h1, batch_r, p0, p1, p2, pb0, pb1, pb2)


def kernel(x, edge_index, batch, W1, b1, g1, be1, W2, b2, g2, be2,
           P0, pb0, P1, pb1, P2, pb2):
    src, dst = edge_index[0], edge_index[1]
    pad = NCHT * CH - EPT
    src_w = jnp.pad(src.reshape(NS, EPT), ((0, 0), (0, pad))).reshape(NS, NCHT, CH)
    dst_w = jnp.pad(dst.reshape(NS, EPT), ((0, 0), (0, pad)),
                    constant_values=N).reshape(NS, NCHT, CH)

    hists = _sc_degree(dst_w)
    dis = _tc_dis(hists)

    hd1 = _tc_hd1(x, W1, dis)
    acc1 = _sc_scatter(hd1, src_w, dst_w)
    h1, hd2 = _tc_layer(acc1, hd1, dis,
                        b1.reshape(1, D), g1.reshape(1, D), be1.reshape(1, D),
                        W2)
    acc2 = _sc_scatter(hd2, src_w, dst_w)

    batch_r = batch.reshape(N // BLK, 1, BLK)
    score = _tc_final(acc2, hd2, dis,
                      b2.reshape(1, D), g2.reshape(1, D), be2.reshape(1, D),
                      x, h1, batch_r,
                      P0, P1, P2,
                      pb0.reshape(1, D_OUT), pb1.reshape(1, D_OUT),
                      pb2.reshape(1, D_OUT))
    return score
